# Initial kernel scaffold; baseline (speedup 1.0000x reference)
import jax
import jax.numpy as jnp
from jax.experimental import pallas as pl


def kernel(z, edge_index, W7, u7, c7, bi7, W8, u8, c8, bi8, W9, u9, c9, bi9, W10, u10, c10, bi10, W11, u11, c11, bi11, W12, u12, c12, bi12, bn7_g, bn7_b, bn8_g, bn8_b, bn9_g, bn9_b, bn10_g, bn10_b):
    raise NotImplementedError("write your pallas kernel here")


# speedup vs baseline: 3.5827x; 3.5827x over previous
import functools

import jax
import jax.numpy as jnp
from jax import lax
from jax.experimental import pallas as pl
from jax.experimental.pallas import tpu as pltpu
from jax.experimental.pallas import tpu_sc as plsc

N = 10000
E = 320000
D = 128
H = 4
HD = H * D

NPA = 10240
RB = 1024
B = 48
NW = 32
ETOT = E + N
CH = -(-ETOT // (NW * B))
EPT = CH * B
ETP = NW * EPT
STRIPE = NPA // 16

_mesh = plsc.VectorSubcoreMesh(core_axis_name="c", subcore_axis_name="s")


def _tables_body(x_ref, w_ref, u_ref, xw_ref, xut_ref):
    x = x_ref[...]
    xw_ref[...] = jnp.dot(x, w_ref[...], preferred_element_type=jnp.float32)
    xut_ref[...] = jnp.dot(x, u_ref[...], preferred_element_type=jnp.float32)


def _tables(x, w, u16):
    return pl.pallas_call(
        _tables_body,
        grid=(NPA // RB,),
        in_specs=[pl.BlockSpec((RB, D), lambda i: (i, 0)),
                  pl.BlockSpec((D, HD), lambda i: (0, 0)),
                  pl.BlockSpec((D, D), lambda i: (0, 0))],
        out_specs=[pl.BlockSpec((RB, HD), lambda i: (i, 0)),
                   pl.BlockSpec((RB, D), lambda i: (i, 0))],
        out_shape=[jax.ShapeDtypeStruct((NPA, HD), jnp.float32),
                   jax.ShapeDtypeStruct((NPA, D), jnp.float32)],
    )(x, w, u16)


def _inv_body(c0_ref, c1_ref, out_ref):
    out_ref[...] = 1.0 / jnp.maximum(c0_ref[...] + c1_ref[...], 1.0)


def _inv_counts(c0, c1):
    return pl.pallas_call(
        _inv_body,
        out_shape=jax.ShapeDtypeStruct((NPA, D), jnp.float32),
    )(c0, c1)


def _bn_body(s0_ref, s1_ref, inv_ref, bi_ref, g_ref, b_ref, out_ref):
    t = (s0_ref[...] + s1_ref[...]) * inv_ref[...] + bi_ref[0:1, :]
    valid = lax.broadcasted_iota(jnp.int32, (NPA, D), 0) < N
    tm = jnp.where(valid, t, 0.0)
    mu = jnp.sum(tm, axis=0, keepdims=True) / N
    var = jnp.sum(tm * tm, axis=0, keepdims=True) / N - mu * mu
    h = jnp.maximum((t - mu) * lax.rsqrt(var + 1e-5) * g_ref[0:1, :]
                    + b_ref[0:1, :], 0.0)
    out_ref[...] = jnp.where(valid, h, 0.0)


def _bn_relu(s0, s1, inv, bi8, g8, b8):
    return pl.pallas_call(
        _bn_body,
        out_shape=jax.ShapeDtypeStruct((NPA, D), jnp.float32),
    )(s0, s1, inv, bi8, g8, b8)


def _mu_body(s0_ref, s1_ref, inv_ref, bi_ref, out_ref):
    t = (s0_ref[...] + s1_ref[...]) * inv_ref[...] + bi_ref[0:1, :]
    valid = lax.broadcasted_iota(jnp.int32, (NPA, D), 0) < N
    out_ref[...] = jnp.where(valid, t, 0.0)


def _mu_out(s0, s1, inv, bi8):
    return pl.pallas_call(
        _mu_body,
        out_shape=jax.ShapeDtypeStruct((NPA, D), jnp.float32),
    )(s0, s1, inv, bi8)


def _lv_body(s0_ref, s1_ref, inv_ref, bi_ref, mu_ref, eps_ref, lv_ref, v_ref):
    t = (s0_ref[...] + s1_ref[...]) * inv_ref[...] + bi_ref[0:1, :]
    valid = lax.broadcasted_iota(jnp.int32, (NPA, D), 0) < N
    lv = jnp.where(valid, t, 0.0)
    lv_ref[...] = lv
    v_ref[...] = eps_ref[...] * jnp.exp(0.5 * lv) + mu_ref[...]


def _lv_v_out(s0, s1, inv, bi8, mu, eps):
    return pl.pallas_call(
        _lv_body,
        out_shape=[jax.ShapeDtypeStruct((NPA, D), jnp.float32),
                   jax.ShapeDtypeStruct((NPA, D), jnp.float32)],
    )(s0, s1, inv, bi8, mu, eps)


@functools.partial(
    pl.kernel,
    mesh=_mesh,
    out_type=jax.ShapeDtypeStruct((2, NPA, D), jnp.float32),
    scratch_types=[
        pltpu.VMEM((B,), jnp.int32),
        pltpu.VMEM((B,), jnp.int32),
        pltpu.VMEM((B, HD), jnp.float32),
        pltpu.VMEM((B, D), jnp.float32),
        pltpu.VMEM((B, D), jnp.float32),
        pltpu.VMEM((16,), jnp.float32),
        pltpu.VMEM((B, D), jnp.float32),
        pltpu.VMEM_SHARED((NPA, D), jnp.float32),
        pltpu.SemaphoreType.DMA,
        pltpu.SemaphoreType.DMA,
        pltpu.SemaphoreType.DMA,
    ],
)
def _edge_kernel(src_hbm, dst_hbm, xw_hbm, xut_hbm, c_hbm, zeros_hbm, out_hbm,
                 src_v, dst_v, xw_v, xus_v, xud_v, c_v, msg_v, acc_sh,
                 sem0, sem1, sem2):
    cid = lax.axis_index("c")
    sid = lax.axis_index("s")
    wid = cid * 16 + sid
    roff = sid * STRIPE
    pltpu.sync_copy(zeros_hbm.at[pl.ds(roff, STRIPE)],
                    acc_sh.at[pl.ds(roff, STRIPE)])
    pltpu.sync_copy(c_hbm, c_v)
    cvec = c_v[...]
    plsc.subcore_barrier()

    def chunk(g, carry):
        ebase = wid * EPT + g * B
        pltpu.sync_copy(src_hbm.at[pl.ds(ebase, B)], src_v)
        pltpu.sync_copy(dst_hbm.at[pl.ds(ebase, B)], dst_v)
        cp0 = pltpu.async_copy(xw_hbm.at[src_v], xw_v, sem0)
        cp1 = pltpu.async_copy(xut_hbm.at[src_v], xus_v, sem1)
        cp2 = pltpu.async_copy(xut_hbm.at[dst_v], xud_v, sem2)
        cp0.wait()
        cp1.wait()
        cp2.wait()

        def edge_body(e, c2):
            d16 = xus_v[e, 0:16] - xud_v[e, 0:16] + cvec
            m = jnp.maximum(jnp.maximum(d16[0], d16[1]),
                            jnp.maximum(d16[2], d16[3]))
            e16 = jnp.exp(d16 - m)
            q16 = e16 / (e16[0] + e16[1] + e16[2] + e16[3])
            q0 = q16[0]
            q1 = q16[1]
            q2 = q16[2]
            q3 = q16[3]
            for f in range(D // 16):
                acc = (q0 * xw_v[e, pl.ds(f * 16, 16)]
                       + q1 * xw_v[e, pl.ds(D + f * 16, 16)]
                       + q2 * xw_v[e, pl.ds(2 * D + f * 16, 16)]
                       + q3 * xw_v[e, pl.ds(3 * D + f * 16, 16)])
                msg_v[e, pl.ds(f * 16, 16)] = acc
            return c2

        lax.fori_loop(0, B, edge_body, 0)
        pltpu.sync_copy(msg_v, acc_sh.at[dst_v], add=True)
        return carry

    lax.fori_loop(0, CH, chunk, 0)
    plsc.subcore_barrier()
    pltpu.sync_copy(acc_sh.at[pl.ds(roff, STRIPE)],
                    out_hbm.at[cid, pl.ds(roff, STRIPE)])


@functools.partial(
    pl.kernel,
    mesh=_mesh,
    out_type=jax.ShapeDtypeStruct((2, NPA, D), jnp.float32),
    scratch_types=[
        pltpu.VMEM((B,), jnp.int32),
        pltpu.VMEM((B, D), jnp.float32),
        pltpu.VMEM_SHARED((NPA, D), jnp.float32),
    ],
)
def _count_kernel(dst_hbm, ones_hbm, zeros_hbm, out_hbm, dst_v, ones_v, acc_sh):
    cid = lax.axis_index("c")
    sid = lax.axis_index("s")
    wid = cid * 16 + sid
    roff = sid * STRIPE
    pltpu.sync_copy(zeros_hbm.at[pl.ds(roff, STRIPE)],
                    acc_sh.at[pl.ds(roff, STRIPE)])
    pltpu.sync_copy(ones_hbm, ones_v)
    plsc.subcore_barrier()

    def chunk(g, carry):
        ebase = wid * EPT + g * B
        pltpu.sync_copy(dst_hbm.at[pl.ds(ebase, B)], dst_v)
        pltpu.sync_copy(ones_v, acc_sh.at[dst_v], add=True)
        return carry

    lax.fori_loop(0, CH, chunk, 0)
    plsc.subcore_barrier()
    pltpu.sync_copy(acc_sh.at[pl.ds(roff, STRIPE)],
                    out_hbm.at[cid, pl.ds(roff, STRIPE)])


def _row8(v):
    return jnp.tile(v[None, :].astype(jnp.float32), (8, 1))


def kernel(z, edge_index, W7, u7, c7, bi7, W8, u8, c8, bi8, W9, u9, c9, bi9,
           W10, u10, c10, bi10, W11, u11, c11, bi11, W12, u12, c12, bi12,
           bn7_g, bn7_b, bn8_g, bn8_b, bn9_g, bn9_b, bn10_g, bn10_b):
    loop = jnp.arange(N, dtype=edge_index.dtype)
    pad = ETP - ETOT
    src_all = jnp.concatenate(
        [edge_index[0], loop, jnp.zeros((pad,), edge_index.dtype)])
    dst_all = jnp.concatenate(
        [edge_index[1], loop, jnp.full((pad,), N, edge_index.dtype)])

    zeros_big = jnp.zeros((NPA, D), jnp.float32)
    ones_blk = jnp.ones((B, D), jnp.float32)

    cnt2 = _count_kernel(dst_all, ones_blk, zeros_big)
    inv = _inv_counts(cnt2[0], cnt2[1])

    layers = [
        (W7, u7, c7, bi7, bn7_g, bn7_b),
        (W8, u8, c8, bi8, bn8_g, bn8_b),
        (W9, u9, c9, bi9, bn9_g, bn9_b),
        (W10, u10, c10, bi10, bn10_g, bn10_b),
    ]

    def upad(u):
        return jnp.concatenate([u, jnp.zeros((D, D - H), jnp.float32)], axis=1)

    def cpad(c):
        return jnp.concatenate([c, jnp.zeros((12,), jnp.float32)])

    h = jnp.zeros((NPA, D), jnp.float32).at[:N].set(z)
    for (W, u, c, bi, g, b) in layers:
        xw, xut = _tables(h, W, upad(u))
        s2 = _edge_kernel(src_all, dst_all, xw, xut, cpad(c), zeros_big)
        h = _bn_relu(s2[0], s2[1], inv, _row8(bi), _row8(g), _row8(b))

    xw, xut = _tables(h, W11, upad(u11))
    s2 = _edge_kernel(src_all, dst_all, xw, xut, cpad(c11), zeros_big)
    mu_full = _mu_out(s2[0], s2[1], inv, _row8(bi11))

    xw, xut = _tables(h, W12, upad(u12))
    s2 = _edge_kernel(src_all, dst_all, xw, xut, cpad(c12), zeros_big)

    eps = jax.random.normal(jax.random.key(42), (N, D), dtype=jnp.float32)
    eps_full = jnp.zeros((NPA, D), jnp.float32).at[:N].set(eps)
    lv_full, v_full = _lv_v_out(s2[0], s2[1], inv, _row8(bi12), mu_full, eps_full)

    return (v_full[:N], mu_full[:N], lv_full[:N])
